# baseline (speedup 1.0000x reference)
import functools

import jax
import jax.numpy as jnp
from jax import lax
from jax.experimental import pallas as pl
from jax.experimental.pallas import tpu as pltpu
from jax.experimental.pallas import tpu_sc as plsc

_INDEX = 42
_T = 2048
_D = 128
_LANES = 16
_NC, _NS = 1, 16
_NW = _NC * _NS
_CHUNK = _T // _NW
_HALO_B = 16
_HALO_F = 48
_BUF = _HALO_B + _CHUNK + _HALO_F
_WINDOWS = (6, 12, 24, 48)
_NWIN = len(_WINDOWS)
_NEG_INF = float("-inf")

_mesh = plsc.VectorSubcoreMesh(
    core_axis_name="c", subcore_axis_name="s",
    num_cores=_NC, num_subcores=_NS)


@functools.partial(
    pl.kernel,
    out_type=(
        jax.ShapeDtypeStruct((_T * _NWIN,), jnp.float32),
        jax.ShapeDtypeStruct((_T,), jnp.int32),
    ),
    mesh=_mesh,
    scratch_types=[
        pltpu.VMEM((_BUF,), jnp.int32),
        pltpu.VMEM((_BUF,), jnp.float32),
        pltpu.VMEM((_BUF,), jnp.int32),
        pltpu.VMEM((_BUF,), jnp.float32),
        pltpu.VMEM((_BUF,), jnp.float32),
        pltpu.VMEM((_CHUNK * _NWIN,), jnp.float32),
        pltpu.VMEM((_CHUNK,), jnp.int32),
        pltpu.SemaphoreType.DMA,
        pltpu.SemaphoreType.DMA,
    ],
)
def _sc_extract(val_hbm, msk_hbm, lv_hbm, lm_hbm,
                idx_v, dat_v, mw_v, nz_v, mval_v,
                lvbuf, lmbuf, sem_a, sem_b):
    wid = lax.axis_index("s") * _NC + lax.axis_index("c")
    base = wid * _CHUNK
    g0 = base - _HALO_B
    iota = lax.iota(jnp.int32, 16)

    def idx_body(k, _):
        g = g0 + k * _LANES + iota
        gc = jnp.clip(g, 0, _T - 1)
        idx_v[pl.ds(k * _LANES, _LANES)] = gc * _D + _INDEX
        return 0
    lax.fori_loop(0, _BUF // _LANES, idx_body, 0)

    cp_a = pltpu.async_copy(val_hbm.at[idx_v], dat_v, sem_a)
    cp_b = pltpu.async_copy(msk_hbm.at[idx_v], mw_v, sem_b)
    cp_a.wait()
    cp_b.wait()

    def _tree_max(xs):
        while len(xs) > 1:
            nxt = [jnp.maximum(xs[i], xs[i + 1]) for i in range(0, len(xs) - 1, 2)]
            if len(xs) % 2:
                nxt.append(xs[-1])
            xs = nxt
        return xs[0]

    def _nz_group(k):
        sl = pl.ds(k * _LANES, _LANES)
        g = g0 + k * _LANES + iota
        v = dat_v[sl]
        valid = (g >= 7) & (g <= _T - 1)
        m = (mw_v[sl] != 0) & valid
        nz_v[sl] = jnp.where(m & (v != 0.0), 1.0, 0.0)
        return m, v

    _nz_group(0)

    def s12_body(k, _):
        l0 = k * _LANES
        m, v = _nz_group(k)
        nb = _tree_max([nz_v[pl.ds(l0 - d, _LANES)] for d in range(1, 13)])
        fm = m & ((v != 0.0) | (nb <= 0.0))
        mval_v[pl.ds(l0, _LANES)] = jnp.where(fm, v, _NEG_INF)
        return 0
    lax.fori_loop(1, _BUF // _LANES, s12_body, 0)

    lane_w = iota & (_NWIN - 1)
    lane_s = lax.shift_right_logical(iota, 2)

    def s3_body(k, _):
        l0 = _HALO_B + k * _LANES

        acc = mval_v[pl.ds(l0 + 1, _LANES)]
        mword = jnp.zeros((16,), jnp.int32)
        vals = []
        dlo = 2
        for wi, w in enumerate(_WINDOWS):
            acc = _tree_max(
                [acc] + [mval_v[pl.ds(l0 + d, _LANES)] for d in range(dlo, w + 1)])
            dlo = w + 1
            got = acc != _NEG_INF
            vals.append(jnp.where(got, acc, 0.0))
            mword = mword | jnp.where(got, 1 << (8 * wi), 0)
        lmbuf[pl.ds(k * _LANES, _LANES)] = mword
        for n in range(_LANES // _NWIN):
            perm = n * _NWIN + lane_s
            t = [v[perm] for v in vals]
            out = jnp.where(lane_w == 0, t[0],
                            jnp.where(lane_w == 1, t[1],
                                      jnp.where(lane_w == 2, t[2], t[3])))
            lvbuf[pl.ds(k * (_LANES * _NWIN) + n * _LANES, _LANES)] = out
        return 0
    lax.fori_loop(0, _CHUNK // _LANES, s3_body, 0)

    pltpu.sync_copy(lvbuf, lv_hbm.at[pl.ds(base * _NWIN, _CHUNK * _NWIN)])
    pltpu.sync_copy(lmbuf, lm_hbm.at[pl.ds(base, _CHUNK)])


def kernel(time, value, mask):
    del time
    val_flat = value.reshape(-1)
    msk_flat = mask.astype(jnp.int32).reshape(-1)
    lv_flat, lm_words = _sc_extract(val_flat, msk_flat)
    lead_value = lv_flat.reshape(_T, _NWIN)
    lead_mask = lax.bitcast_convert_type(lm_words, jnp.uint8).view(jnp.bool_)
    return lead_value, lead_mask


# speedup vs baseline: 1.0047x; 1.0047x over previous
import functools

import jax
import jax.numpy as jnp
from jax import lax
from jax.experimental import pallas as pl
from jax.experimental.pallas import tpu as pltpu
from jax.experimental.pallas import tpu_sc as plsc

_INDEX = 42
_T = 2048
_D = 128
_LANES = 16
_NC, _NS = 1, 16
_NW = _NC * _NS
_CHUNK = _T // _NW
_HALO_B = 16
_HALO_F = 48
_BUF = _HALO_B + _CHUNK + _HALO_F
_WINDOWS = (6, 12, 24, 48)
_NWIN = len(_WINDOWS)
_NEG_INF = float("-inf")

_mesh = plsc.VectorSubcoreMesh(
    core_axis_name="c", subcore_axis_name="s",
    num_cores=_NC, num_subcores=_NS)


@functools.partial(
    pl.kernel,
    out_type=(
        jax.ShapeDtypeStruct((_T * _NWIN,), jnp.float32),
        jax.ShapeDtypeStruct((_T,), jnp.int32),
    ),
    mesh=_mesh,
    scratch_types=[
        pltpu.VMEM((_BUF,), jnp.int32),
        pltpu.VMEM((_BUF,), jnp.float32),
        pltpu.VMEM((_BUF,), jnp.int32),
        pltpu.VMEM((_BUF,), jnp.float32),
        pltpu.VMEM((_BUF,), jnp.float32),
        pltpu.VMEM((_CHUNK * _NWIN,), jnp.float32),
        pltpu.VMEM((_CHUNK,), jnp.int32),
        pltpu.SemaphoreType.DMA,
        pltpu.SemaphoreType.DMA,
    ],
)
def _sc_extract(val_hbm, msk_hbm, lv_hbm, lm_hbm,
                idx_v, dat_v, mw_v, nz_v, mval_v,
                lvbuf, lmbuf, sem_a, sem_b):
    wid = lax.axis_index("s") * _NC + lax.axis_index("c")
    base = wid * _CHUNK
    g0 = base - _HALO_B
    iota = lax.iota(jnp.int32, 16)

    def idx_body(k, _):
        g = g0 + k * _LANES + iota
        gc = jnp.clip(g, 0, _T - 1)
        idx_v[pl.ds(k * _LANES, _LANES)] = gc * _D + _INDEX
        return 0
    lax.fori_loop(0, _BUF // _LANES, idx_body, 0)

    cp_a = pltpu.async_copy(val_hbm.at[idx_v], dat_v, sem_a)
    cp_b = pltpu.async_copy(msk_hbm.at[idx_v], mw_v, sem_b)
    cp_a.wait()
    cp_b.wait()

    def _tree_max(xs):
        while len(xs) > 1:
            nxt = [jnp.maximum(xs[i], xs[i + 1]) for i in range(0, len(xs) - 1, 2)]
            if len(xs) % 2:
                nxt.append(xs[-1])
            xs = nxt
        return xs[0]

    def _nz_group(k):
        sl = pl.ds(k * _LANES, _LANES)
        g = g0 + k * _LANES + iota
        v = dat_v[sl]
        valid = (g >= 7) & (g <= _T - 1)
        m = (mw_v[sl] != 0) & valid
        nz_v[sl] = jnp.where(m & (v != 0.0), 1.0, 0.0)
        return m, v

    _nz_group(0)

    def s12_body(k, _):
        l0 = k * _LANES
        m, v = _nz_group(k)
        nb = _tree_max([nz_v[pl.ds(l0 - d, _LANES)] for d in range(1, 13)])
        fm = m & ((v != 0.0) | (nb <= 0.0))
        mval_v[pl.ds(l0, _LANES)] = jnp.where(fm, v, _NEG_INF)
        return 0
    lax.fori_loop(1, _BUF // _LANES, s12_body, 0)

    lane_w = iota & (_NWIN - 1)
    lane_s = lax.shift_right_logical(iota, 2)

    def s3_body(k, _):
        l0 = _HALO_B + k * _LANES

        acc = mval_v[pl.ds(l0 + 1, _LANES)]
        mword = jnp.zeros((16,), jnp.int32)
        vals = []
        dlo = 2
        for wi, w in enumerate(_WINDOWS):
            acc = _tree_max(
                [acc] + [mval_v[pl.ds(l0 + d, _LANES)] for d in range(dlo, w + 1)])
            dlo = w + 1
            got = acc != _NEG_INF
            vals.append(jnp.where(got, acc, 0.0))
            mword = mword | jnp.where(got, 1 << (8 * wi), 0)
        lmbuf[pl.ds(k * _LANES, _LANES)] = mword
        for n in range(_LANES // _NWIN):
            perm = n * _NWIN + lane_s
            t = [v[perm] for v in vals]
            out = jnp.where(lane_w == 0, t[0],
                            jnp.where(lane_w == 1, t[1],
                                      jnp.where(lane_w == 2, t[2], t[3])))
            lvbuf[pl.ds(k * (_LANES * _NWIN) + n * _LANES, _LANES)] = out
        return 0
    lax.fori_loop(0, _CHUNK // _LANES, s3_body, 0)

    pltpu.sync_copy(lvbuf, lv_hbm.at[pl.ds(base * _NWIN, _CHUNK * _NWIN)])
    pltpu.sync_copy(lmbuf, lm_hbm.at[pl.ds(base, _CHUNK)])


def kernel(time, value, mask):
    del time
    val_flat = value.reshape(-1)
    msk_flat = mask.astype(jnp.int32).reshape(-1)
    lv_flat, lm_words = _sc_extract(val_flat, msk_flat)
    lead_value = lv_flat.reshape(_T, _NWIN)
    lead_mask = lax.bitcast_convert_type(lm_words, jnp.uint8) != 0
    return lead_value, lead_mask
